# baseline (speedup 1.0000x reference)
import functools

import jax
import jax.numpy as jnp
from jax import lax
from jax.experimental import pallas as pl
from jax.experimental.pallas import tpu as pltpu
from jax.experimental.pallas import tpu_sc as plsc

_VOCAB = 50304
_HIDDEN = 768
_N_EMB = 6
_TOTAL = 4 * 2048

_NC = 2
_NS = 16
_NW = _NC * _NS
_PER_W = _TOTAL // _NW
_CHUNK = 64
_NCHUNK = _PER_W // _CHUNK


@functools.cache
def _build():
    mesh = plsc.VectorSubcoreMesh(core_axis_name="c", subcore_axis_name="s")

    @functools.partial(
        pl.kernel,
        mesh=mesh,
        out_type=jax.ShapeDtypeStruct((_TOTAL, _HIDDEN), jnp.float32),
        scratch_types=[
            pltpu.VMEM((_PER_W,), jnp.int32),
            pltpu.VMEM((_CHUNK, _HIDDEN), jnp.float32),
            pltpu.VMEM((_CHUNK, _HIDDEN), jnp.float32),
            pltpu.SemaphoreType.DMA,
            pltpu.SemaphoreType.DMA,
            pltpu.SemaphoreType.DMA,
            pltpu.SemaphoreType.DMA,
        ],
    )
    def _gather1(idx_hbm, w, o, idx_v, buf0, buf1, g0, g1, s0, s1):
        wid = lax.axis_index("s") * _NC + lax.axis_index("c")
        base = wid * _PER_W
        pltpu.sync_copy(idx_hbm.at[pl.ds(base, _PER_W)], idx_v)
        bufs, gsems, ssems = (buf0, buf1), (g0, g1), (s0, s1)

        def start_gather(c):
            p = c % 2
            return pltpu.async_copy(
                w.at[idx_v.at[pl.ds(c * _CHUNK, _CHUNK)]], bufs[p], gsems[p])

        def start_write(c):
            p = c % 2
            return pltpu.async_copy(
                bufs[p], o.at[pl.ds(base + c * _CHUNK, _CHUNK)], ssems[p])

        writes = [None, None]
        gather = start_gather(0)
        for c in range(_NCHUNK):
            p = c % 2
            nxt = None
            if c + 1 < _NCHUNK:
                pn = (c + 1) % 2
                if writes[pn] is not None:
                    writes[pn].wait()
                nxt = start_gather(c + 1)
            gather.wait()
            writes[p] = start_write(c)
            gather = nxt
        for wr in writes:
            if wr is not None:
                wr.wait()

    return _gather1


def kernel(inputs, W0, W1, W2, W3, W4, W5):
    B, S = inputs.shape
    idx = inputs.reshape(-1).astype(jnp.int32)
    g = _build()
    ve = [g(idx, W).reshape(B, S, _HIDDEN)
          for W in (W0, W1, W2, W3, W4, W5)]
    return tuple(ve + ve[::-1])


# speedup vs baseline: 1.3502x; 1.3502x over previous
import functools

import jax
import jax.numpy as jnp
from jax import lax
from jax.experimental import pallas as pl
from jax.experimental.pallas import tpu as pltpu
from jax.experimental.pallas import tpu_sc as plsc

_VOCAB = 50304
_HIDDEN = 768
_N_EMB = 6
_TOTAL = 4 * 2048

_NC = 2
_NS = 16
_NW = _NC * _NS
_PER_W = _TOTAL // _NW
_CHUNK = 64
_NCHUNK = _PER_W // _CHUNK


@functools.cache
def _build():
    mesh = plsc.VectorSubcoreMesh(core_axis_name="c", subcore_axis_name="s")

    @functools.partial(
        pl.kernel,
        mesh=mesh,
        out_type=[jax.ShapeDtypeStruct((_TOTAL, _HIDDEN), jnp.float32)]
        * (2 * _N_EMB),
        scratch_types=[
            pltpu.VMEM((_PER_W,), jnp.int32),
            pltpu.VMEM((_CHUNK, _HIDDEN), jnp.float32),
            pltpu.VMEM((_CHUNK, _HIDDEN), jnp.float32),
            pltpu.SemaphoreType.DMA,
            pltpu.SemaphoreType.DMA,
            pltpu.SemaphoreType.DMA,
            pltpu.SemaphoreType.DMA,
        ],
    )
    def _gather6(idx_hbm, w0, w1, w2, w3, w4, w5,
                 o0, o1, o2, o3, o4, o5, o6, o7, o8, o9, o10, o11,
                 idx_v, buf0, buf1, g0, g1, s0, s1):
        wid = lax.axis_index("s") * _NC + lax.axis_index("c")
        base = wid * _PER_W
        pltpu.sync_copy(idx_hbm.at[pl.ds(base, _PER_W)], idx_v)
        bufs, gsems, ssems = (buf0, buf1), (g0, g1), (s0, s1)
        outs = (o0, o1, o2, o3, o4, o5, o6, o7, o8, o9, o10, o11)
        ws = (w0, w1, w2, w3, w4, w5)
        steps = [(ws[t], outs[t], outs[11 - t], c)
                 for t in range(_N_EMB) for c in range(_NCHUNK)]

        def start_gather(i):
            w, _, _, c = steps[i]
            p = i % 2
            return pltpu.async_copy(
                w.at[idx_v.at[pl.ds(c * _CHUNK, _CHUNK)]], bufs[p], gsems[p])

        def start_writes(i):
            _, o_lo, o_hi, c = steps[i]
            p = i % 2
            dst = pl.ds(base + c * _CHUNK, _CHUNK)
            return [pltpu.async_copy(bufs[p], o_lo.at[dst], ssems[p]),
                    pltpu.async_copy(bufs[p], o_hi.at[dst], ssems[p])]

        writes = [[], []]
        gather = start_gather(0)
        for i in range(len(steps)):
            p = i % 2
            nxt = None
            if i + 1 < len(steps):
                pn = (i + 1) % 2
                for wr in writes[pn]:
                    wr.wait()
                nxt = start_gather(i + 1)
            gather.wait()
            writes[p] = start_writes(i)
            gather = nxt
        for wl in writes:
            for wr in wl:
                wr.wait()

    return _gather6


def kernel(inputs, W0, W1, W2, W3, W4, W5):
    B, S = inputs.shape
    idx = inputs.reshape(-1).astype(jnp.int32)
    outs = _build()(idx, W0, W1, W2, W3, W4, W5)
    return tuple(o.reshape(B, S, _HIDDEN) for o in outs)


# speedup vs baseline: 1.3582x; 1.0059x over previous
import functools

import jax
import jax.numpy as jnp
from jax import lax
from jax.experimental import pallas as pl
from jax.experimental.pallas import tpu as pltpu
from jax.experimental.pallas import tpu_sc as plsc

_VOCAB = 50304
_HIDDEN = 768
_N_EMB = 6
_B = 4
_S = 2048
_TOTAL = _B * _S

_NC = 2
_NS = 16
_NW = _NC * _NS
_PER_W = _TOTAL // _NW
_CHUNK = 64
_NCHUNK = _PER_W // _CHUNK


@functools.cache
def _build():
    mesh = plsc.VectorSubcoreMesh(core_axis_name="c", subcore_axis_name="s")

    @functools.partial(
        pl.kernel,
        mesh=mesh,
        out_type=[jax.ShapeDtypeStruct((_TOTAL, _HIDDEN), jnp.float32)]
        * (2 * _N_EMB),
        scratch_types=[
            pltpu.VMEM((_PER_W,), jnp.int32),
            pltpu.VMEM((_CHUNK, _HIDDEN), jnp.float32),
            pltpu.VMEM((_CHUNK, _HIDDEN), jnp.float32),
            pltpu.SemaphoreType.DMA,
            pltpu.SemaphoreType.DMA,
            pltpu.SemaphoreType.DMA,
            pltpu.SemaphoreType.DMA,
        ],
    )
    def _gather6(idx_hbm, w0, w1, w2, w3, w4, w5,
                 o0, o1, o2, o3, o4, o5, o6, o7, o8, o9, o10, o11,
                 idx_v, buf0, buf1, g0, g1, s0, s1):
        wid = lax.axis_index("s") * _NC + lax.axis_index("c")
        base = wid * _PER_W
        per_row = _S // _PER_W
        pltpu.sync_copy(
            idx_hbm.at[wid // per_row,
                       pl.ds((wid % per_row) * _PER_W, _PER_W)], idx_v)
        bufs, gsems, ssems = (buf0, buf1), (g0, g1), (s0, s1)
        outs = (o0, o1, o2, o3, o4, o5, o6, o7, o8, o9, o10, o11)
        ws = (w0, w1, w2, w3, w4, w5)
        steps = [(ws[t], outs[t], outs[11 - t], c)
                 for t in range(_N_EMB) for c in range(_NCHUNK)]

        def start_gather(i):
            w, _, _, c = steps[i]
            p = i % 2
            return pltpu.async_copy(
                w.at[idx_v.at[pl.ds(c * _CHUNK, _CHUNK)]], bufs[p], gsems[p])

        def start_writes(i):
            _, o_lo, o_hi, c = steps[i]
            p = i % 2
            dst = pl.ds(base + c * _CHUNK, _CHUNK)
            return [pltpu.async_copy(bufs[p], o_lo.at[dst], ssems[p]),
                    pltpu.async_copy(bufs[p], o_hi.at[dst], ssems[p])]

        writes = [[], []]
        gather = start_gather(0)
        for i in range(len(steps)):
            p = i % 2
            nxt = None
            if i + 1 < len(steps):
                pn = (i + 1) % 2
                for wr in writes[pn]:
                    wr.wait()
                nxt = start_gather(i + 1)
            gather.wait()
            writes[p] = start_writes(i)
            gather = nxt
        for wl in writes:
            for wr in wl:
                wr.wait()

    return _gather6


def kernel(inputs, W0, W1, W2, W3, W4, W5):
    B, S = inputs.shape
    outs = _build()(inputs.astype(jnp.int32), W0, W1, W2, W3, W4, W5)
    return tuple(o.reshape(B, S, _HIDDEN) for o in outs)


# speedup vs baseline: 1.3591x; 1.0007x over previous
import functools

import jax
import jax.numpy as jnp
from jax import lax
from jax.experimental import pallas as pl
from jax.experimental.pallas import tpu as pltpu
from jax.experimental.pallas import tpu_sc as plsc

_VOCAB = 50304
_HIDDEN = 768
_N_EMB = 6
_B = 4
_S = 2048
_TOTAL = _B * _S

_NC = 2
_NS = 16
_NW = _NC * _NS
_PER_W = _TOTAL // _NW
_CHUNK = 64
_NCHUNK = _PER_W // _CHUNK


@functools.cache
def _build():
    mesh = plsc.VectorSubcoreMesh(core_axis_name="c", subcore_axis_name="s")

    @functools.partial(
        pl.kernel,
        mesh=mesh,
        out_type=[jax.ShapeDtypeStruct((_TOTAL, _HIDDEN), jnp.float32)]
        * (2 * _N_EMB),
        scratch_types=[
            pltpu.VMEM((_PER_W,), jnp.int32),
            pltpu.VMEM((_CHUNK, _HIDDEN), jnp.float32),
            pltpu.VMEM((_CHUNK, _HIDDEN), jnp.float32),
            pltpu.SemaphoreType.DMA,
            pltpu.SemaphoreType.DMA,
            pltpu.SemaphoreType.DMA,
            pltpu.SemaphoreType.DMA,
        ],
    )
    def _gather6(idx_hbm, w0, w1, w2, w3, w4, w5,
                 o0, o1, o2, o3, o4, o5, o6, o7, o8, o9, o10, o11,
                 idx_v, buf0, buf1, g0, g1, s0, s1):
        wid = lax.axis_index("s") * _NC + lax.axis_index("c")
        base = wid * _PER_W
        per_row = _S // _PER_W
        pltpu.sync_copy(
            idx_hbm.at[wid // per_row,
                       pl.ds((wid % per_row) * _PER_W, _PER_W)], idx_v)
        bufs, gsems, ssems = (buf0, buf1), (g0, g1), (s0, s1)
        outs = (o0, o1, o2, o3, o4, o5, o6, o7, o8, o9, o10, o11)
        ws = (w0, w1, w2, w3, w4, w5)
        first = ((0, 16), (16, 48), (64, 64), (128, 64), (192, 64))
        mid = ((0, 64), (64, 64), (128, 64), (192, 64))
        last = ((0, 64), (64, 64), (128, 64), (192, 32), (224, 16), (240, 16))
        tbl_chunks = (first, mid, mid, mid, mid, last)
        steps = [(ws[t], outs[t], outs[11 - t], off, sz)
                 for t in range(_N_EMB) for off, sz in tbl_chunks[t]]

        def start_gather(i):
            w, _, _, off, sz = steps[i]
            p = i % 2
            return pltpu.async_copy(
                w.at[idx_v.at[pl.ds(off, sz)]],
                bufs[p].at[pl.ds(0, sz)], gsems[p])

        def start_writes(i):
            _, o_lo, o_hi, off, sz = steps[i]
            p = i % 2
            src = bufs[p].at[pl.ds(0, sz)]
            dst = pl.ds(base + off, sz)
            return [pltpu.async_copy(src, o_lo.at[dst], ssems[p]),
                    pltpu.async_copy(src, o_hi.at[dst], ssems[p])]

        writes = [[], []]
        gather = start_gather(0)
        for i in range(len(steps)):
            p = i % 2
            nxt = None
            if i + 1 < len(steps):
                pn = (i + 1) % 2
                for wr in writes[pn]:
                    wr.wait()
                nxt = start_gather(i + 1)
            gather.wait()
            writes[p] = start_writes(i)
            gather = nxt
        for wl in writes:
            for wr in wl:
                wr.wait()

    return _gather6


def kernel(inputs, W0, W1, W2, W3, W4, W5):
    B, S = inputs.shape
    outs = _build()(inputs.astype(jnp.int32), W0, W1, W2, W3, W4, W5)
    return tuple(o.reshape(B, S, _HIDDEN) for o in outs)


# speedup vs baseline: 1.3594x; 1.0002x over previous
import functools

import jax
import jax.numpy as jnp
from jax import lax
from jax.experimental import pallas as pl
from jax.experimental.pallas import tpu as pltpu
from jax.experimental.pallas import tpu_sc as plsc

_VOCAB = 50304
_HIDDEN = 768
_N_EMB = 6
_B = 4
_S = 2048
_TOTAL = _B * _S

_NC = 2
_NS = 16
_NW = _NC * _NS
_PER_W = _TOTAL // _NW
_CHUNK = 64
_NCHUNK = _PER_W // _CHUNK


@functools.cache
def _build():
    mesh = plsc.VectorSubcoreMesh(core_axis_name="c", subcore_axis_name="s")

    @functools.partial(
        pl.kernel,
        mesh=mesh,
        out_type=[jax.ShapeDtypeStruct((_TOTAL, _HIDDEN), jnp.float32)]
        * (2 * _N_EMB),
        scratch_types=[
            pltpu.VMEM((_PER_W,), jnp.int32),
            pltpu.VMEM((_CHUNK, _HIDDEN), jnp.float32),
            pltpu.VMEM((_CHUNK, _HIDDEN), jnp.float32),
            pltpu.SemaphoreType.DMA,
            pltpu.SemaphoreType.DMA,
            pltpu.SemaphoreType.DMA,
            pltpu.SemaphoreType.DMA,
        ],
    )
    def _gather6(idx_hbm, w0, w1, w2, w3, w4, w5,
                 o0, o1, o2, o3, o4, o5, o6, o7, o8, o9, o10, o11,
                 idx_v, buf0, buf1, g0, g1, s0, s1):
        wid = lax.axis_index("s") * _NC + lax.axis_index("c")
        base = wid * _PER_W
        per_row = _S // _PER_W
        pltpu.sync_copy(
            idx_hbm.at[wid // per_row,
                       pl.ds((wid % per_row) * _PER_W, _PER_W)], idx_v)
        bufs, gsems, ssems = (buf0, buf1), (g0, g1), (s0, s1)
        outs = (o0, o1, o2, o3, o4, o5, o6, o7, o8, o9, o10, o11)
        ws = (w0, w1, w2, w3, w4, w5)
        steps = [(ws[t], outs[t], outs[11 - t], c)
                 for t in range(_N_EMB) for c in range(_NCHUNK)]

        def start_gather(i):
            w, _, _, c = steps[i]
            p = i % 2
            return pltpu.async_copy(
                w.at[idx_v.at[pl.ds(c * _CHUNK, _CHUNK)]], bufs[p], gsems[p])

        def start_writes(i):
            _, o_lo, o_hi, c = steps[i]
            p = i % 2
            dst = pl.ds(base + c * _CHUNK, _CHUNK)
            return [pltpu.async_copy(bufs[p], o_lo.at[dst], ssems[p]),
                    pltpu.async_copy(bufs[p], o_hi.at[dst], ssems[p])]

        writes = [[], []]
        gather = start_gather(0)
        for i in range(len(steps)):
            p = i % 2
            nxt = None
            if i + 1 < len(steps):
                pn = (i + 1) % 2
                for wr in writes[pn]:
                    wr.wait()
                nxt = start_gather(i + 1)
            gather.wait()
            writes[p] = start_writes(i)
            gather = nxt
        for wl in writes:
            for wr in wl:
                wr.wait()

    return _gather6


def kernel(inputs, W0, W1, W2, W3, W4, W5):
    B, S = inputs.shape
    outs = _build()(inputs.astype(jnp.int32), W0, W1, W2, W3, W4, W5)
    return tuple(o.reshape(B, S, _HIDDEN) for o in outs)
